# baseline (device time: 27039 ns/iter reference)
import jax
import jax.numpy as jnp
from jax import lax
from jax.experimental import pallas as pl
from jax.experimental.pallas import tpu as pltpu

N_DEV = 8
B, SQ, DM = 2, 256, 768
HQ_PER = 8
DH = 64
DQ_PER = HQ_PER * DH
DKV = 2 * DH
ROWS = B * SQ
CHUNK = ROWS // N_DEV
CPB = SQ // CHUNK


def kernel(x, Wq, Wo, Wk, Wv):
    my_pos = lax.axis_index("i")
    xb = x.astype(jnp.bfloat16).reshape(ROWS, DM)
    wqb = Wq.astype(jnp.bfloat16)
    wob = Wo.astype(jnp.bfloat16)
    wkb = lax.dynamic_index_in_dim(
        Wk.reshape(DM, N_DEV, DKV), my_pos, 1, keepdims=False
    ).astype(jnp.bfloat16)
    wvb = lax.dynamic_index_in_dim(
        Wv.reshape(DM, N_DEV, DKV), my_pos, 1, keepdims=False
    ).astype(jnp.bfloat16)

    def body(x_ref, wq_ref, wk_ref, wv_ref, wo_ref, out_ref,
             partial_ref, buf1, buf2, send_sems1, recv_sems1,
             send_sems2, recv_sems2):
        my = lax.axis_index("i")

        barrier_sem = pltpu.get_barrier_semaphore()
        for k in range(1, N_DEV):
            pl.semaphore_signal(
                barrier_sem, inc=1,
                device_id=(lax.rem(my + k, N_DEV),),
                device_id_type=pl.DeviceIdType.MESH,
            )

        xm = x_ref[...]
        qb = jnp.dot(xm, wq_ref[...],
                     preferred_element_type=jnp.float32).astype(jnp.bfloat16)
        kb = jnp.dot(xm, wk_ref[...],
                     preferred_element_type=jnp.float32).astype(jnp.bfloat16)
        vb = jnp.dot(xm, wv_ref[...],
                     preferred_element_type=jnp.float32).astype(jnp.bfloat16)

        def half_chunk_descs(bi):
            descs = []
            for c in range(bi * CPB, (bi + 1) * CPB):
                descs.append(pltpu.make_async_remote_copy(
                    src_ref=partial_ref.at[pl.ds(c * CHUNK, CHUNK)],
                    dst_ref=buf1.at[my],
                    send_sem=send_sems1.at[c],
                    recv_sem=recv_sems1.at[my],
                    device_id=(c,),
                    device_id_type=pl.DeviceIdType.MESH,
                ))
            return descs

        def compute_half(bi):
            row_blocks = []
            for hh in range(HQ_PER):
                g = hh // 4
                qs = qb[bi * SQ:(bi + 1) * SQ, hh * DH:(hh + 1) * DH]
                ks = kb[bi * SQ:(bi + 1) * SQ, g * DH:(g + 1) * DH]
                vs = vb[bi * SQ:(bi + 1) * SQ, g * DH:(g + 1) * DH]
                s = jnp.dot(qs, ks.T, preferred_element_type=jnp.float32) * 0.125
                p = jnp.exp(s)
                l = jnp.sum(p, axis=-1, keepdims=True)
                o = jnp.dot(p.astype(jnp.bfloat16), vs,
                            preferred_element_type=jnp.float32) / l
                row_blocks.append(o.astype(jnp.bfloat16))
            attn_b = jnp.concatenate(row_blocks, axis=1)
            cp = jnp.dot(attn_b, wo_ref[...], preferred_element_type=jnp.float32)
            partial_ref[bi * SQ:(bi + 1) * SQ, :] = cp.astype(jnp.bfloat16)

        def send_half(bi):
            for c, rdma in zip(range(bi * CPB, (bi + 1) * CPB),
                               half_chunk_descs(bi)):
                @pl.when(my != c)
                def _(rdma=rdma):
                    rdma.start()

        def halves(first, second):
            compute_half(first)
            pl.semaphore_wait(barrier_sem, N_DEV - 1)
            send_half(first)
            compute_half(second)
            send_half(second)

        @pl.when(my < CPB)
        def _():
            halves(1, 0)

        @pl.when(my >= CPB)
        def _():
            halves(0, 1)

        for s in range(N_DEV):
            @pl.when(my == s)
            def _(s=s):
                buf1[s] = partial_ref[s * CHUNK:(s + 1) * CHUNK, :]

        red = None
        for s in range(N_DEV):
            recv = pltpu.make_async_remote_copy(
                src_ref=partial_ref.at[pl.ds(s * CHUNK, CHUNK)],
                dst_ref=buf1.at[s],
                send_sem=send_sems1.at[s],
                recv_sem=recv_sems1.at[s],
                device_id=(s,),
                device_id_type=pl.DeviceIdType.MESH,
            )
            @pl.when(my != s)
            def _(recv=recv):
                recv.wait_recv()
            contrib = buf1[s].astype(jnp.float32)
            red = contrib if red is None else red + contrib
        buf2[pl.ds(my * CHUNK, CHUNK), :] = red.astype(jnp.bfloat16)

        p2 = []
        for c in range(N_DEV):
            rdma = pltpu.make_async_remote_copy(
                src_ref=buf2.at[pl.ds(my * CHUNK, CHUNK)],
                dst_ref=buf2.at[pl.ds(my * CHUNK, CHUNK)],
                send_sem=send_sems2.at[c],
                recv_sem=recv_sems2.at[my],
                device_id=(c,),
                device_id_type=pl.DeviceIdType.MESH,
            )
            @pl.when(my != c)
            def _(rdma=rdma):
                rdma.start()
            p2.append(rdma)

        for s in range(N_DEV):
            recv = pltpu.make_async_remote_copy(
                src_ref=buf2.at[pl.ds(s * CHUNK, CHUNK)],
                dst_ref=buf2.at[pl.ds(s * CHUNK, CHUNK)],
                send_sem=send_sems2.at[s],
                recv_sem=recv_sems2.at[s],
                device_id=(s,),
                device_id_type=pl.DeviceIdType.MESH,
            )
            @pl.when(my != s)
            def _(recv=recv):
                recv.wait_recv()

        out_ref[...] = buf2[...].astype(jnp.float32).reshape(B, SQ, DM)

        for c in range(N_DEV):
            send1 = pltpu.make_async_remote_copy(
                src_ref=partial_ref.at[pl.ds(c * CHUNK, CHUNK)],
                dst_ref=buf1.at[c],
                send_sem=send_sems1.at[c],
                recv_sem=recv_sems1.at[c],
                device_id=(c,),
                device_id_type=pl.DeviceIdType.MESH,
            )
            @pl.when(my != c)
            def _(send1=send1, p2c=p2[c]):
                send1.wait_send()
                p2c.wait_send()

    return pl.pallas_call(
        body,
        out_shape=jax.ShapeDtypeStruct((B, SQ, DM), jnp.float32),
        in_specs=[pl.BlockSpec(memory_space=pltpu.VMEM)] * 5,
        out_specs=pl.BlockSpec(memory_space=pltpu.VMEM),
        scratch_shapes=[
            pltpu.VMEM((ROWS, DM), jnp.bfloat16),
            pltpu.VMEM((N_DEV, CHUNK, DM), jnp.bfloat16),
            pltpu.VMEM((ROWS, DM), jnp.bfloat16),
            pltpu.SemaphoreType.DMA((N_DEV,)),
            pltpu.SemaphoreType.DMA((N_DEV,)),
            pltpu.SemaphoreType.DMA((N_DEV,)),
            pltpu.SemaphoreType.DMA((N_DEV,)),
        ],
        compiler_params=pltpu.CompilerParams(collective_id=0),
    )(xb, wqb, wkb, wvb, wob)


# device time: 24855 ns/iter; 1.0879x vs baseline; 1.0879x over previous
import jax
import jax.numpy as jnp
from jax import lax
from jax.experimental import pallas as pl
from jax.experimental.pallas import tpu as pltpu

N_DEV = 8
B, SQ, DM = 2, 256, 768
HQ_PER = 8
DH = 64
DQ_PER = HQ_PER * DH
DKV = 2 * DH
ROWS = B * SQ
CHUNK = ROWS // N_DEV
CPB = SQ // CHUNK


def kernel(x, Wq, Wo, Wk, Wv):
    my_pos = lax.axis_index("i")
    xb = x.astype(jnp.bfloat16).reshape(ROWS, DM)
    wqb = Wq.astype(jnp.bfloat16)
    wob = Wo.astype(jnp.bfloat16)
    wkb = lax.dynamic_slice_in_dim(Wk, my_pos * DKV, DKV, 1).astype(jnp.bfloat16)
    wvb = lax.dynamic_slice_in_dim(Wv, my_pos * DKV, DKV, 1).astype(jnp.bfloat16)

    def body(x_ref, wq_ref, wk_ref, wv_ref, wo_ref, out_ref,
             partial_ref, buf1, buf2, send_sems1, recv_sems1,
             send_sems2, recv_sems2):
        my = lax.axis_index("i")

        barrier_sem = pltpu.get_barrier_semaphore()
        for k in range(1, N_DEV):
            pl.semaphore_signal(
                barrier_sem, inc=1,
                device_id=(lax.rem(my + k, N_DEV),),
                device_id_type=pl.DeviceIdType.MESH,
            )

        xm = x_ref[...]
        qb = jnp.dot(xm, wq_ref[...],
                     preferred_element_type=jnp.float32).astype(jnp.bfloat16)
        kb = jnp.dot(xm, wk_ref[...],
                     preferred_element_type=jnp.float32).astype(jnp.bfloat16)
        vb = jnp.dot(xm, wv_ref[...],
                     preferred_element_type=jnp.float32).astype(jnp.bfloat16)

        def half_chunk_descs(bi):
            descs = []
            for c in range(bi * CPB, (bi + 1) * CPB):
                descs.append(pltpu.make_async_remote_copy(
                    src_ref=partial_ref.at[pl.ds(c * CHUNK, CHUNK)],
                    dst_ref=buf1.at[my],
                    send_sem=send_sems1.at[c],
                    recv_sem=recv_sems1.at[my],
                    device_id=(c,),
                    device_id_type=pl.DeviceIdType.MESH,
                ))
            return descs

        def compute_half(bi):
            row_blocks = []
            for hh in range(HQ_PER):
                g = hh // 4
                qs = qb[bi * SQ:(bi + 1) * SQ, hh * DH:(hh + 1) * DH]
                ks = kb[bi * SQ:(bi + 1) * SQ, g * DH:(g + 1) * DH]
                vs = vb[bi * SQ:(bi + 1) * SQ, g * DH:(g + 1) * DH]
                s = jnp.dot(qs, ks.T, preferred_element_type=jnp.float32) * 0.125
                p = jnp.exp(s)
                l = jnp.sum(p, axis=-1, keepdims=True)
                o = jnp.dot(p.astype(jnp.bfloat16), vs,
                            preferred_element_type=jnp.float32) / l
                row_blocks.append(o.astype(jnp.bfloat16))
            attn_b = jnp.concatenate(row_blocks, axis=1)
            cp = jnp.dot(attn_b, wo_ref[...], preferred_element_type=jnp.float32)
            partial_ref[bi * SQ:(bi + 1) * SQ, :] = cp.astype(jnp.bfloat16)

        def send_half(bi):
            for c, rdma in zip(range(bi * CPB, (bi + 1) * CPB),
                               half_chunk_descs(bi)):
                @pl.when(my != c)
                def _(rdma=rdma):
                    rdma.start()

        def halves(first, second):
            compute_half(first)
            pl.semaphore_wait(barrier_sem, N_DEV - 1)
            send_half(first)
            compute_half(second)
            send_half(second)

        @pl.when(my < CPB)
        def _():
            halves(1, 0)

        @pl.when(my >= CPB)
        def _():
            halves(0, 1)

        for s in range(N_DEV):
            @pl.when(my == s)
            def _(s=s):
                buf1[s] = partial_ref[s * CHUNK:(s + 1) * CHUNK, :]

        red = None
        for s in range(N_DEV):
            recv = pltpu.make_async_remote_copy(
                src_ref=partial_ref.at[pl.ds(s * CHUNK, CHUNK)],
                dst_ref=buf1.at[s],
                send_sem=send_sems1.at[s],
                recv_sem=recv_sems1.at[s],
                device_id=(s,),
                device_id_type=pl.DeviceIdType.MESH,
            )
            @pl.when(my != s)
            def _(recv=recv):
                recv.wait_recv()
            contrib = buf1[s].astype(jnp.float32)
            red = contrib if red is None else red + contrib
        buf2[pl.ds(my * CHUNK, CHUNK), :] = red.astype(jnp.bfloat16)

        p2 = []
        for c in range(N_DEV):
            rdma = pltpu.make_async_remote_copy(
                src_ref=buf2.at[pl.ds(my * CHUNK, CHUNK)],
                dst_ref=buf2.at[pl.ds(my * CHUNK, CHUNK)],
                send_sem=send_sems2.at[c],
                recv_sem=recv_sems2.at[my],
                device_id=(c,),
                device_id_type=pl.DeviceIdType.MESH,
            )
            @pl.when(my != c)
            def _(rdma=rdma):
                rdma.start()
            p2.append(rdma)

        for s in range(N_DEV):
            recv = pltpu.make_async_remote_copy(
                src_ref=buf2.at[pl.ds(s * CHUNK, CHUNK)],
                dst_ref=buf2.at[pl.ds(s * CHUNK, CHUNK)],
                send_sem=send_sems2.at[s],
                recv_sem=recv_sems2.at[s],
                device_id=(s,),
                device_id_type=pl.DeviceIdType.MESH,
            )
            @pl.when(my != s)
            def _(recv=recv):
                recv.wait_recv()

        out_ref[...] = buf2[...].astype(jnp.float32).reshape(B, SQ, DM)

        for c in range(N_DEV):
            send1 = pltpu.make_async_remote_copy(
                src_ref=partial_ref.at[pl.ds(c * CHUNK, CHUNK)],
                dst_ref=buf1.at[c],
                send_sem=send_sems1.at[c],
                recv_sem=recv_sems1.at[c],
                device_id=(c,),
                device_id_type=pl.DeviceIdType.MESH,
            )
            @pl.when(my != c)
            def _(send1=send1, p2c=p2[c]):
                send1.wait_send()
                p2c.wait_send()

    return pl.pallas_call(
        body,
        out_shape=jax.ShapeDtypeStruct((B, SQ, DM), jnp.float32),
        in_specs=[pl.BlockSpec(memory_space=pltpu.VMEM)] * 5,
        out_specs=pl.BlockSpec(memory_space=pltpu.VMEM),
        scratch_shapes=[
            pltpu.VMEM((ROWS, DM), jnp.bfloat16),
            pltpu.VMEM((N_DEV, CHUNK, DM), jnp.bfloat16),
            pltpu.VMEM((ROWS, DM), jnp.bfloat16),
            pltpu.SemaphoreType.DMA((N_DEV,)),
            pltpu.SemaphoreType.DMA((N_DEV,)),
            pltpu.SemaphoreType.DMA((N_DEV,)),
            pltpu.SemaphoreType.DMA((N_DEV,)),
        ],
        compiler_params=pltpu.CompilerParams(collective_id=0),
    )(xb, wqb, wkb, wvb, wob)


# device time: 24826 ns/iter; 1.0891x vs baseline; 1.0012x over previous
import jax
import jax.numpy as jnp
from jax import lax
from jax.experimental import pallas as pl
from jax.experimental.pallas import tpu as pltpu

N_DEV = 8
B, SQ, DM = 2, 256, 768
HQ_PER = 8
DH = 64
DQ_PER = HQ_PER * DH
DKV = 2 * DH
ROWS = B * SQ
CHUNK = ROWS // N_DEV
CPB = SQ // CHUNK


def kernel(x, Wq, Wo, Wk, Wv):
    my_pos = lax.axis_index("i")
    xb = x.reshape(ROWS, DM)
    wqb = Wq.astype(jnp.bfloat16)
    wob = Wo.astype(jnp.bfloat16)
    wkb = lax.dynamic_slice_in_dim(Wk, my_pos * DKV, DKV, 1).astype(jnp.bfloat16)
    wvb = lax.dynamic_slice_in_dim(Wv, my_pos * DKV, DKV, 1).astype(jnp.bfloat16)

    def body(x_ref, wq_ref, wk_ref, wv_ref, wo_ref, out_ref,
             partial_ref, buf1, buf2, send_sems1, recv_sems1,
             send_sems2, recv_sems2):
        my = lax.axis_index("i")

        barrier_sem = pltpu.get_barrier_semaphore()
        for k in range(1, N_DEV):
            pl.semaphore_signal(
                barrier_sem, inc=1,
                device_id=(lax.rem(my + k, N_DEV),),
                device_id_type=pl.DeviceIdType.MESH,
            )

        xm = x_ref[...].astype(jnp.bfloat16)
        qb = jnp.dot(xm, wq_ref[...],
                     preferred_element_type=jnp.float32).astype(jnp.bfloat16)
        kb = jnp.dot(xm, wk_ref[...],
                     preferred_element_type=jnp.float32).astype(jnp.bfloat16)
        vb = jnp.dot(xm, wv_ref[...],
                     preferred_element_type=jnp.float32).astype(jnp.bfloat16)

        def half_chunk_descs(bi):
            descs = []
            for c in range(bi * CPB, (bi + 1) * CPB):
                descs.append(pltpu.make_async_remote_copy(
                    src_ref=partial_ref.at[pl.ds(c * CHUNK, CHUNK)],
                    dst_ref=buf1.at[my],
                    send_sem=send_sems1.at[c],
                    recv_sem=recv_sems1.at[my],
                    device_id=(c,),
                    device_id_type=pl.DeviceIdType.MESH,
                ))
            return descs

        def compute_half(bi):
            row_blocks = []
            for hh in range(HQ_PER):
                g = hh // 4
                qs = qb[bi * SQ:(bi + 1) * SQ, hh * DH:(hh + 1) * DH]
                ks = kb[bi * SQ:(bi + 1) * SQ, g * DH:(g + 1) * DH]
                vs = vb[bi * SQ:(bi + 1) * SQ, g * DH:(g + 1) * DH]
                s = jnp.dot(qs, ks.T, preferred_element_type=jnp.float32) * 0.125
                p = jnp.exp(s)
                l = jnp.sum(p, axis=-1, keepdims=True)
                o = jnp.dot(p.astype(jnp.bfloat16), vs,
                            preferred_element_type=jnp.float32) / l
                row_blocks.append(o.astype(jnp.bfloat16))
            attn_b = jnp.concatenate(row_blocks, axis=1)
            cp = jnp.dot(attn_b, wo_ref[...], preferred_element_type=jnp.float32)
            partial_ref[bi * SQ:(bi + 1) * SQ, :] = cp.astype(jnp.bfloat16)

        def send_half(bi):
            for c, rdma in zip(range(bi * CPB, (bi + 1) * CPB),
                               half_chunk_descs(bi)):
                @pl.when(my != c)
                def _(rdma=rdma):
                    rdma.start()

        def halves(first, second):
            compute_half(first)
            pl.semaphore_wait(barrier_sem, N_DEV - 1)
            send_half(first)
            compute_half(second)
            send_half(second)

        @pl.when(my < CPB)
        def _():
            halves(1, 0)

        @pl.when(my >= CPB)
        def _():
            halves(0, 1)

        for s in range(N_DEV):
            @pl.when(my == s)
            def _(s=s):
                buf1[s] = partial_ref[s * CHUNK:(s + 1) * CHUNK, :]

        red = None
        for s in range(N_DEV):
            recv = pltpu.make_async_remote_copy(
                src_ref=partial_ref.at[pl.ds(s * CHUNK, CHUNK)],
                dst_ref=buf1.at[s],
                send_sem=send_sems1.at[s],
                recv_sem=recv_sems1.at[s],
                device_id=(s,),
                device_id_type=pl.DeviceIdType.MESH,
            )
            @pl.when(my != s)
            def _(recv=recv):
                recv.wait_recv()
            contrib = buf1[s].astype(jnp.float32)
            red = contrib if red is None else red + contrib
        buf2[pl.ds(my * CHUNK, CHUNK), :] = red.astype(jnp.bfloat16)

        p2 = []
        for c in range(N_DEV):
            rdma = pltpu.make_async_remote_copy(
                src_ref=buf2.at[pl.ds(my * CHUNK, CHUNK)],
                dst_ref=buf2.at[pl.ds(my * CHUNK, CHUNK)],
                send_sem=send_sems2.at[c],
                recv_sem=recv_sems2.at[my],
                device_id=(c,),
                device_id_type=pl.DeviceIdType.MESH,
            )
            @pl.when(my != c)
            def _(rdma=rdma):
                rdma.start()
            p2.append(rdma)

        for s in range(N_DEV):
            recv = pltpu.make_async_remote_copy(
                src_ref=buf2.at[pl.ds(s * CHUNK, CHUNK)],
                dst_ref=buf2.at[pl.ds(s * CHUNK, CHUNK)],
                send_sem=send_sems2.at[s],
                recv_sem=recv_sems2.at[s],
                device_id=(s,),
                device_id_type=pl.DeviceIdType.MESH,
            )
            @pl.when(my != s)
            def _(recv=recv):
                recv.wait_recv()

        out_ref[...] = buf2[...].astype(jnp.float32).reshape(B, SQ, DM)

        for c in range(N_DEV):
            send1 = pltpu.make_async_remote_copy(
                src_ref=partial_ref.at[pl.ds(c * CHUNK, CHUNK)],
                dst_ref=buf1.at[c],
                send_sem=send_sems1.at[c],
                recv_sem=recv_sems1.at[c],
                device_id=(c,),
                device_id_type=pl.DeviceIdType.MESH,
            )
            @pl.when(my != c)
            def _(send1=send1, p2c=p2[c]):
                send1.wait_send()
                p2c.wait_send()

    return pl.pallas_call(
        body,
        out_shape=jax.ShapeDtypeStruct((B, SQ, DM), jnp.float32),
        in_specs=[pl.BlockSpec(memory_space=pltpu.VMEM)] * 5,
        out_specs=pl.BlockSpec(memory_space=pltpu.VMEM),
        scratch_shapes=[
            pltpu.VMEM((ROWS, DM), jnp.bfloat16),
            pltpu.VMEM((N_DEV, CHUNK, DM), jnp.bfloat16),
            pltpu.VMEM((ROWS, DM), jnp.bfloat16),
            pltpu.SemaphoreType.DMA((N_DEV,)),
            pltpu.SemaphoreType.DMA((N_DEV,)),
            pltpu.SemaphoreType.DMA((N_DEV,)),
            pltpu.SemaphoreType.DMA((N_DEV,)),
        ],
        compiler_params=pltpu.CompilerParams(collective_id=0),
    )(xb, wqb, wkb, wvb, wob)


# device time: 24666 ns/iter; 1.0962x vs baseline; 1.0065x over previous
import jax
import jax.numpy as jnp
from jax import lax
from jax.experimental import pallas as pl
from jax.experimental.pallas import tpu as pltpu

N_DEV = 8
B, SQ, DM = 2, 256, 768
HQ_PER = 8
DH = 64
DQ_PER = HQ_PER * DH
DKV = 2 * DH
ROWS = B * SQ
CHUNK = ROWS // N_DEV
CPB = SQ // CHUNK


def kernel(x, Wq, Wo, Wk, Wv):
    my_pos = lax.axis_index("i")
    xb = x.reshape(ROWS, DM)
    wqb = Wq.astype(jnp.bfloat16)
    wob = Wo.astype(jnp.bfloat16)
    wkb = lax.dynamic_slice_in_dim(Wk, my_pos * DKV, DKV, 1).astype(jnp.bfloat16)
    wvb = lax.dynamic_slice_in_dim(Wv, my_pos * DKV, DKV, 1).astype(jnp.bfloat16)

    def body(x_ref, wq_ref, wk_ref, wv_ref, wo_ref, out_ref,
             partial_ref, buf1, buf2, send_sems1, recv_sems1,
             send_sems2, recv_sems2):
        my = lax.axis_index("i")

        barrier_sem = pltpu.get_barrier_semaphore()
        for k in range(1, N_DEV):
            pl.semaphore_signal(
                barrier_sem, inc=1,
                device_id=(lax.rem(my + k, N_DEV),),
                device_id_type=pl.DeviceIdType.MESH,
            )

        xm = x_ref[...].astype(jnp.bfloat16)
        qb = jnp.dot(xm, wq_ref[...],
                     preferred_element_type=jnp.float32).astype(jnp.bfloat16)
        kb = jnp.dot(xm, wk_ref[...],
                     preferred_element_type=jnp.float32).astype(jnp.bfloat16)
        vb = jnp.dot(xm, wv_ref[...],
                     preferred_element_type=jnp.float32).astype(jnp.bfloat16)

        def half_chunk_descs(bi):
            descs = []
            for c in range(bi * CPB, (bi + 1) * CPB):
                descs.append(pltpu.make_async_remote_copy(
                    src_ref=partial_ref.at[pl.ds(c * CHUNK, CHUNK)],
                    dst_ref=buf1.at[my],
                    send_sem=send_sems1.at[c],
                    recv_sem=recv_sems1.at[my],
                    device_id=(c,),
                    device_id_type=pl.DeviceIdType.MESH,
                ))
            return descs

        def compute_half(bi):
            row_blocks = []
            for hh in range(HQ_PER):
                g = hh // 4
                qs = qb[bi * SQ:(bi + 1) * SQ, hh * DH:(hh + 1) * DH]
                ks = kb[bi * SQ:(bi + 1) * SQ, g * DH:(g + 1) * DH]
                vs = vb[bi * SQ:(bi + 1) * SQ, g * DH:(g + 1) * DH]
                s = jnp.dot(qs, ks.T, preferred_element_type=jnp.float32) * 0.125
                p = jnp.exp(s)
                l = jnp.sum(p, axis=-1, keepdims=True)
                o = jnp.dot(p.astype(jnp.bfloat16), vs,
                            preferred_element_type=jnp.float32) / l
                row_blocks.append(o.astype(jnp.bfloat16))
            attn_b = jnp.concatenate(row_blocks, axis=1)
            cp = jnp.dot(attn_b, wo_ref[...], preferred_element_type=jnp.float32)
            partial_ref[bi * SQ:(bi + 1) * SQ, :] = cp.astype(jnp.bfloat16)

        def send_half(bi):
            for c, rdma in zip(range(bi * CPB, (bi + 1) * CPB),
                               half_chunk_descs(bi)):
                @pl.when(my != c)
                def _(rdma=rdma):
                    rdma.start()

        def halves(first, second):
            compute_half(first)
            pl.semaphore_wait(barrier_sem, N_DEV - 1)
            send_half(first)
            compute_half(second)
            send_half(second)

        @pl.when(my < CPB)
        def _():
            halves(1, 0)

        @pl.when(my >= CPB)
        def _():
            halves(0, 1)

        for s in range(N_DEV):
            @pl.when(my == s)
            def _(s=s):
                buf1[s] = partial_ref[s * CHUNK:(s + 1) * CHUNK, :]

        red = None
        for s in range(N_DEV):
            recv = pltpu.make_async_remote_copy(
                src_ref=partial_ref.at[pl.ds(s * CHUNK, CHUNK)],
                dst_ref=buf1.at[s],
                send_sem=send_sems1.at[s],
                recv_sem=recv_sems1.at[s],
                device_id=(s,),
                device_id_type=pl.DeviceIdType.MESH,
            )
            @pl.when(my != s)
            def _(recv=recv):
                recv.wait_recv()
            contrib = buf1[s].astype(jnp.float32)
            red = contrib if red is None else red + contrib
        buf2[pl.ds(my * CHUNK, CHUNK), :] = red.astype(jnp.bfloat16)

        p2 = []
        for c in range(N_DEV):
            rdma = pltpu.make_async_remote_copy(
                src_ref=buf2.at[pl.ds(my * CHUNK, CHUNK)],
                dst_ref=buf2.at[pl.ds(my * CHUNK, CHUNK)],
                send_sem=send_sems2.at[c],
                recv_sem=recv_sems2.at[my],
                device_id=(c,),
                device_id_type=pl.DeviceIdType.MESH,
            )
            @pl.when(my != c)
            def _(rdma=rdma):
                rdma.start()
            p2.append(rdma)

        for s in range(N_DEV):
            recv = pltpu.make_async_remote_copy(
                src_ref=buf2.at[pl.ds(s * CHUNK, CHUNK)],
                dst_ref=buf2.at[pl.ds(s * CHUNK, CHUNK)],
                send_sem=send_sems2.at[s],
                recv_sem=recv_sems2.at[s],
                device_id=(s,),
                device_id_type=pl.DeviceIdType.MESH,
            )
            @pl.when(my != s)
            def _(recv=recv):
                recv.wait_recv()
            bi, q0 = divmod(s * CHUNK, SQ)
            out_ref[bi, q0:q0 + CHUNK, :] = (
                buf2[s * CHUNK:(s + 1) * CHUNK, :].astype(jnp.float32))

        for c in range(N_DEV):
            send1 = pltpu.make_async_remote_copy(
                src_ref=partial_ref.at[pl.ds(c * CHUNK, CHUNK)],
                dst_ref=buf1.at[c],
                send_sem=send_sems1.at[c],
                recv_sem=recv_sems1.at[c],
                device_id=(c,),
                device_id_type=pl.DeviceIdType.MESH,
            )
            @pl.when(my != c)
            def _(send1=send1, p2c=p2[c]):
                send1.wait_send()
                p2c.wait_send()

    return pl.pallas_call(
        body,
        out_shape=jax.ShapeDtypeStruct((B, SQ, DM), jnp.float32),
        in_specs=[pl.BlockSpec(memory_space=pltpu.VMEM)] * 5,
        out_specs=pl.BlockSpec(memory_space=pltpu.VMEM),
        scratch_shapes=[
            pltpu.VMEM((ROWS, DM), jnp.bfloat16),
            pltpu.VMEM((N_DEV, CHUNK, DM), jnp.bfloat16),
            pltpu.VMEM((ROWS, DM), jnp.bfloat16),
            pltpu.SemaphoreType.DMA((N_DEV,)),
            pltpu.SemaphoreType.DMA((N_DEV,)),
            pltpu.SemaphoreType.DMA((N_DEV,)),
            pltpu.SemaphoreType.DMA((N_DEV,)),
        ],
        compiler_params=pltpu.CompilerParams(collective_id=0),
    )(xb, wqb, wkb, wvb, wob)
